# baseline (speedup 1.0000x reference)
import functools
import math

import jax
import jax.numpy as jnp
from jax import lax
from jax.experimental import pallas as pl
from jax.experimental.pallas import tpu as pltpu
from jax.experimental.pallas import tpu_sc as plsc

N = 10000
E = 320000
D = 128
OUT_DIM = 64

NC = 2
NS = 16
NW = NC * NS
CH = 128
NPAD = 10240
NPT = NPAD // NS
E_PAD = ((E + NW * CH * 4 - 1) // (NW * CH * 4)) * (NW * CH * 4)
EPT = E_PAD // NW
NCH = EPT // CH
RB = NPAD // 128
FCH = 64

NEG = float(jnp.finfo(jnp.float32).min)

_MESH = plsc.VectorSubcoreMesh(core_axis_name="c", subcore_axis_name="s")
_SC_PARAMS = pltpu.CompilerParams(needs_layout_passes=False)


def _zero_vec(ref, n):
    for t in range(n // 16):
        ref[pl.ds(t * 16, 16)] = jnp.zeros((16,), jnp.float32)


@functools.partial(
    pl.kernel,
    out_type=[
        jax.ShapeDtypeStruct((NC, 2, NPAD), jnp.float32),
        jax.ShapeDtypeStruct((E_PAD,), jnp.int32),
        jax.ShapeDtypeStruct((E_PAD,), jnp.int32),
        jax.ShapeDtypeStruct((NW, 16), jnp.int32),
    ],
    mesh=_MESH,
    compiler_params=_SC_PARAMS,
    scratch_types=[
        pltpu.VMEM((NPAD,), jnp.float32),
        pltpu.VMEM((2, CH), jnp.int32),
        pltpu.VMEM((2, CH), jnp.int32),
        pltpu.VMEM((2, CH), jnp.int32),
        pltpu.VMEM((2, CH), jnp.int32),
        pltpu.VMEM((2, CH), jnp.float32),
        pltpu.VMEM((2, CH), jnp.float32),
        pltpu.VMEM((2, CH), jnp.float32),
        pltpu.VMEM((2, CH), jnp.float32),
        pltpu.VMEM((EPT + 16,), jnp.int32),
        pltpu.VMEM((EPT + 16,), jnp.int32),
        pltpu.VMEM((16,), jnp.int32),
        pltpu.VMEM((16,), jnp.int32),
        pltpu.VMEM((NPT,), jnp.float32),
        pltpu.VMEM_SHARED((NPAD,), jnp.float32),
        pltpu.VMEM_SHARED((NPAD,), jnp.float32),
        pltpu.SemaphoreType.DMA,
        pltpu.SemaphoreType.DMA,
    ],
)
def _sc_degc(mask_hbm, src_hbm, dst_hbm, cnt_hbm,
             out_hbm, csrc_hbm, cdst_hbm, cnt_out_hbm,
             table_v, sidx0, didx0, sidx1, didx1, vout0, vin0, vout1, vin1,
             csrc_v, cdst_v, cobuf, cibuf, zbuf, odeg_sh, ideg_sh, sem0, sem1):
    c = lax.axis_index("c")
    s = lax.axis_index("s")
    wid = s * NC + c
    pltpu.sync_copy(mask_hbm, table_v)
    pltpu.sync_copy(cnt_hbm.at[wid], cibuf)
    _zero_vec(zbuf, NPT)
    pltpu.sync_copy(zbuf, odeg_sh.at[pl.ds(s * NPT, NPT)])
    pltpu.sync_copy(zbuf, ideg_sh.at[pl.ds(s * NPT, NPT)])
    plsc.subcore_barrier()

    rows_w = cibuf[pl.ds(0, 16)][0]
    G = rows_w // 4
    base_e = wid * EPT

    def load(sidx, didx, row, sem):
        for r in range(2):
            off = base_e + (row + r) * CH
            pltpu.async_copy(src_hbm.at[pl.ds(off, CH)], sidx.at[r], sem)
            pltpu.async_copy(dst_hbm.at[pl.ds(off, CH)], didx.at[r], sem)

    def drain(sidx, didx, sem):
        for r in range(2):
            pltpu.make_async_copy(src_hbm.at[pl.ds(0, CH)], sidx.at[r],
                                  sem).wait()
            pltpu.make_async_copy(dst_hbm.at[pl.ds(0, CH)], didx.at[r],
                                  sem).wait()

    def work(sidx, didx, vout, vin, cnt):
        for r in range(2):
            for t in range(CH // 16):
                sv = sidx[r, pl.ds(t * 16, 16)]
                dv = didx[r, pl.ds(t * 16, 16)]
                vout[r, pl.ds(t * 16, 16)] = plsc.load_gather(table_v, [dv])
                vin[r, pl.ds(t * 16, 16)] = plsc.load_gather(table_v, [sv])
        for r in range(2):
            pltpu.sync_copy(vout.at[r], odeg_sh.at[sidx.at[r]], add=True)
            pltpu.sync_copy(vin.at[r], ideg_sh.at[didx.at[r]], add=True)
        for r in range(2):
            for t in range(CH // 16):
                sv = sidx[r, pl.ds(t * 16, 16)]
                dv = didx[r, pl.ds(t * 16, 16)]
                m = (vout[r, pl.ds(t * 16, 16)]
                     * vin[r, pl.ds(t * 16, 16)]) > 0.0
                plsc.store_compressed(csrc_v.at[pl.ds(cnt, 16)], sv, mask=m)
                plsc.store_compressed(cdst_v.at[pl.ds(cnt, 16)], dv, mask=m)
                cnt = cnt + jnp.sum(m.astype(jnp.int32))
        return cnt

    @pl.when(G >= 1)
    def _():
        load(sidx0, didx0, 0, sem0)

    def body(g, cnt):
        j0 = 4 * g
        load(sidx1, didx1, j0 + 2, sem1)
        drain(sidx0, didx0, sem0)
        cnt = work(sidx0, didx0, vout0, vin0, cnt)

        @pl.when(g + 1 < G)
        def _():
            load(sidx0, didx0, j0 + 4, sem0)

        drain(sidx1, didx1, sem1)
        cnt = work(sidx1, didx1, vout1, vin1, cnt)
        return cnt

    cnt = lax.fori_loop(0, G, body, jnp.int32(0))

    padv = jnp.int32(N) + lax.iota(jnp.int32, 16)
    target = ((cnt + 4 * CH - 1) // (4 * CH)) * (4 * CH)

    def padbody(t, cc):
        @pl.when(cc < target)
        def _():
            csrc_v[pl.ds(cc, 16)] = padv
            cdst_v[pl.ds(cc, 16)] = padv
        return cc + 16

    lax.fori_loop(0, 4 * CH // 16, padbody, cnt)
    cobuf[...] = jnp.broadcast_to(target // CH, (16,)).astype(jnp.int32)
    pltpu.sync_copy(cobuf, cnt_out_hbm.at[wid])
    pltpu.sync_copy(csrc_v.at[pl.ds(0, EPT)], csrc_hbm.at[pl.ds(base_e, EPT)])
    pltpu.sync_copy(cdst_v.at[pl.ds(0, EPT)], cdst_hbm.at[pl.ds(base_e, EPT)])

    plsc.subcore_barrier()
    pltpu.sync_copy(odeg_sh.at[pl.ds(s * NPT, NPT)], zbuf)
    pltpu.sync_copy(zbuf, out_hbm.at[c, 0, pl.ds(s * NPT, NPT)])
    pltpu.sync_copy(ideg_sh.at[pl.ds(s * NPT, NPT)], zbuf)
    pltpu.sync_copy(zbuf, out_hbm.at[c, 1, pl.ds(s * NPT, NPT)])


@functools.partial(
    pl.kernel,
    out_type=jax.ShapeDtypeStruct((NC, NPAD, D), jnp.float32),
    mesh=_MESH,
    compiler_params=_SC_PARAMS,
    scratch_types=[
        pltpu.VMEM((1, FCH), jnp.int32),
        pltpu.VMEM((1, FCH), jnp.int32),
        pltpu.VMEM((1, FCH), jnp.int32),
        pltpu.VMEM((1, FCH), jnp.int32),
        pltpu.VMEM((1, FCH), jnp.int32),
        pltpu.VMEM((1, FCH), jnp.int32),
        pltpu.VMEM((FCH, D), jnp.float32),
        pltpu.VMEM((FCH, D), jnp.float32),
        pltpu.VMEM((FCH, D), jnp.float32),
        pltpu.VMEM((16,), jnp.int32),
        pltpu.VMEM_SHARED((NPAD, D), jnp.float32),
        pltpu.SemaphoreType.DMA,
        pltpu.SemaphoreType.DMA,
        pltpu.SemaphoreType.DMA,
        pltpu.SemaphoreType.DMA,
        pltpu.SemaphoreType.DMA,
        pltpu.SemaphoreType.DMA,
    ],
)
def _sc_featmp(h_hbm, src_hbm, dst_hbm, cnt_hbm, out_hbm,
               si0, si1, si2, di0, di1, di2, rw0, rw1, rw2, cibuf,
               agg_sh, sg0, sg1, sg2, ss0, ss1, ss2):
    c = lax.axis_index("c")
    s = lax.axis_index("s")
    wid = s * NC + c
    pltpu.sync_copy(cnt_hbm.at[wid], cibuf)
    SI = [si0, si1, si2]
    DI = [di0, di1, di2]
    RW = [rw0, rw1, rw2]
    SG = [sg0, sg1, sg2]
    SS = [ss0, ss1, ss2]

    def zrow(r, carry):
        for t in range(D // 16):
            rw0[r, pl.ds(t * 16, 16)] = jnp.zeros((16,), jnp.float32)
        return carry

    lax.fori_loop(0, FCH, zrow, 0)
    for q in range(NPT // FCH):
        pltpu.sync_copy(rw0, agg_sh.at[pl.ds(s * NPT + q * FCH, FCH)])
    plsc.subcore_barrier()

    nfc = cibuf[pl.ds(0, 16)][0] * (CH // FCH)
    G3 = (nfc + 2) // 3
    base_e = wid * EPT

    def wait_scatter(b):
        pltpu.make_async_copy(RW[b], agg_sh.at[DI[b].at[0]], SS[b]).wait()

    def body(g, carry):
        for b in range(3):
            j = 3 * g + b

            @pl.when(j < nfc)
            def _(b=b, j=j):
                @pl.when(g >= 1)
                def _():
                    wait_scatter(b)
                off = base_e + j * FCH
                pltpu.sync_copy(src_hbm.at[pl.ds(off, FCH)], SI[b].at[0])
                pltpu.sync_copy(dst_hbm.at[pl.ds(off, FCH)], DI[b].at[0])
                pltpu.async_copy(h_hbm.at[SI[b].at[0]], RW[b], SG[b])

        for b in range(3):
            j = 3 * g + b

            @pl.when(j < nfc)
            def _(b=b, j=j):
                pltpu.make_async_copy(h_hbm.at[SI[b].at[0]], RW[b],
                                      SG[b]).wait()
                pltpu.async_copy(RW[b], agg_sh.at[DI[b].at[0]], SS[b],
                                 add=True)

        return carry

    lax.fori_loop(0, G3, body, 0)
    for b in range(3):
        @pl.when(nfc > b)
        def _(b=b):
            wait_scatter(b)

    plsc.subcore_barrier()
    for q in range(NPT // FCH):
        pltpu.sync_copy(agg_sh.at[pl.ds(s * NPT + q * FCH, FCH)], rw0)
        pltpu.sync_copy(rw0, out_hbm.at[c, pl.ds(s * NPT + q * FCH, FCH)])


@functools.partial(
    pl.kernel,
    out_type=jax.ShapeDtypeStruct((NC, NPAD), jnp.float32),
    mesh=_MESH,
    compiler_params=_SC_PARAMS,
    scratch_types=[
        pltpu.VMEM((NPAD,), jnp.float32),
        pltpu.VMEM((2, CH), jnp.int32),
        pltpu.VMEM((2, CH), jnp.int32),
        pltpu.VMEM((2, CH), jnp.int32),
        pltpu.VMEM((2, CH), jnp.int32),
        pltpu.VMEM((2, CH), jnp.float32),
        pltpu.VMEM((2, CH), jnp.float32),
        pltpu.VMEM((16,), jnp.int32),
        pltpu.VMEM((NPT,), jnp.float32),
        pltpu.VMEM_SHARED((NPAD,), jnp.float32),
        pltpu.SemaphoreType.DMA,
        pltpu.SemaphoreType.DMA,
    ],
)
def _sc_scalarmp(val_hbm, src_hbm, dst_hbm, cnt_hbm, out_hbm,
                 table_v, sidx0, didx0, sidx1, didx1, vals0, vals1,
                 cibuf, zbuf, acc_sh, sem0, sem1):
    c = lax.axis_index("c")
    s = lax.axis_index("s")
    wid = s * NC + c
    pltpu.sync_copy(val_hbm, table_v)
    pltpu.sync_copy(cnt_hbm.at[wid], cibuf)
    _zero_vec(zbuf, NPT)
    pltpu.sync_copy(zbuf, acc_sh.at[pl.ds(s * NPT, NPT)])
    plsc.subcore_barrier()

    rows_w = cibuf[pl.ds(0, 16)][0]
    G = rows_w // 4
    base_e = wid * EPT

    def load(sidx, didx, row, sem):
        for r in range(2):
            off = base_e + (row + r) * CH
            pltpu.async_copy(src_hbm.at[pl.ds(off, CH)], sidx.at[r], sem)
            pltpu.async_copy(dst_hbm.at[pl.ds(off, CH)], didx.at[r], sem)

    def drain(sidx, didx, sem):
        for r in range(2):
            pltpu.make_async_copy(src_hbm.at[pl.ds(0, CH)], sidx.at[r],
                                  sem).wait()
            pltpu.make_async_copy(dst_hbm.at[pl.ds(0, CH)], didx.at[r],
                                  sem).wait()

    def work(sidx, didx, vals):
        for r in range(2):
            for t in range(CH // 16):
                sv = sidx[r, pl.ds(t * 16, 16)]
                vals[r, pl.ds(t * 16, 16)] = plsc.load_gather(table_v, [sv])
        for r in range(2):
            pltpu.sync_copy(vals.at[r], acc_sh.at[didx.at[r]], add=True)

    @pl.when(G >= 1)
    def _():
        load(sidx0, didx0, 0, sem0)

    def body(g, carry):
        j0 = 4 * g
        load(sidx1, didx1, j0 + 2, sem1)
        drain(sidx0, didx0, sem0)
        work(sidx0, didx0, vals0)

        @pl.when(g + 1 < G)
        def _():
            load(sidx0, didx0, j0 + 4, sem0)

        drain(sidx1, didx1, sem1)
        work(sidx1, didx1, vals1)
        return carry

    lax.fori_loop(0, G, body, 0)
    plsc.subcore_barrier()
    pltpu.sync_copy(acc_sh.at[pl.ds(s * NPT, NPT)], zbuf)
    pltpu.sync_copy(zbuf, out_hbm.at[c, pl.ds(s * NPT, NPT)])


def _norm_body(od0, od1, id0, id1, mask, feat, h, innorm, onorm):
    od = (od0[...] + od1[...]) * mask[...]
    idg = (id0[...] + id1[...]) * mask[...]
    on = jnp.where(od > 0, lax.rsqrt(jnp.maximum(od, 1e-12)), 0.0)
    inn = jnp.where(idg > 0, lax.rsqrt(jnp.maximum(idg, 1e-12)), 0.0)
    onorm[...] = on
    innorm[...] = inn
    h[...] = feat[...] * on


def _tc_norm(od0, od1, id0, id1, mask, feat):
    return pl.pallas_call(
        _norm_body,
        out_shape=[
            jax.ShapeDtypeStruct((NPAD, D), jnp.float32),
            jax.ShapeDtypeStruct((NPAD, 1), jnp.float32),
            jax.ShapeDtypeStruct((NPAD, 1), jnp.float32),
        ],
    )(od0, od1, id0, id1, mask, feat)


def _conv_body(agg0, agg1, innorm, onorm, W, b, sW, out, q):
    z = (agg0[...] + agg1[...]) * innorm[...]
    o = jnp.maximum(jnp.dot(z, W[...], preferred_element_type=jnp.float32)
                    + b[...], 0.0)
    out[...] = o
    q[...] = jnp.dot(o, sW[...], preferred_element_type=jnp.float32) * onorm[...]


def _tc_conv(agg0, agg1, innorm, onorm, W, b, sW):
    GB = 1280
    g = NPAD // GB
    return pl.pallas_call(
        _conv_body,
        grid=(g,),
        in_specs=[
            pl.BlockSpec((GB, D), lambda i: (i, 0)),
            pl.BlockSpec((GB, D), lambda i: (i, 0)),
            pl.BlockSpec((GB, 1), lambda i: (i, 0)),
            pl.BlockSpec((GB, 1), lambda i: (i, 0)),
            pl.BlockSpec((D, D), lambda i: (0, 0)),
            pl.BlockSpec((1, D), lambda i: (0, 0)),
            pl.BlockSpec((D, 1), lambda i: (0, 0)),
        ],
        out_specs=[
            pl.BlockSpec((GB, D), lambda i: (i, 0)),
            pl.BlockSpec((GB, 1), lambda i: (i, 0)),
        ],
        out_shape=[
            jax.ShapeDtypeStruct((NPAD, D), jnp.float32),
            jax.ShapeDtypeStruct((NPAD, 1), jnp.float32),
        ],
    )(agg0, agg1, innorm, onorm, W, b, sW)


def _pool_a_body(k, sp0, sp1, innorm, sb, mask, nm_out, tm_out):
    score = (sp0[...] + sp1[...]) * innorm[...] + sb[0, 0]
    sm = jnp.where(mask[...] > 0, score, NEG)
    bits = lax.bitcast_convert_type(sm, jnp.int32)
    ikey = jnp.where(bits >= 0, bits, bits ^ jnp.int32(0x7FFFFFFF))
    cnt_nn = jnp.sum((ikey >= 0).astype(jnp.int32))
    x0 = jnp.where(cnt_nn >= k, jnp.int32(0), jnp.int32(-2147483648))

    def body(bit, xx):
        y = xx | (jnp.int32(1) << (30 - bit))
        cnt = jnp.sum((ikey >= y).astype(jnp.int32))
        return jnp.where(cnt >= k, y, xx)

    T = lax.fori_loop(0, 31, body, x0)
    cnt_gt = jnp.sum((ikey > T).astype(jnp.int32))
    need = (k - cnt_gt).astype(jnp.float32)
    eqf = (ikey == T).astype(jnp.float32)
    ia = lax.broadcasted_iota(jnp.int32, (D, D), 0)
    ib = lax.broadcasted_iota(jnp.int32, (D, D), 1)
    ltri = (ia <= ib).astype(jnp.float32)
    ra = lax.broadcasted_iota(jnp.int32, (RB, RB), 0)
    rb = lax.broadcasted_iota(jnp.int32, (RB, RB), 1)
    stri = (rb < ra).astype(jnp.float32)
    inrow = jnp.dot(eqf, ltri, preferred_element_type=jnp.float32)
    rowtot = jnp.sum(eqf, axis=1, keepdims=True)
    rowpref = jnp.dot(stri, rowtot, preferred_element_type=jnp.float32)
    rank = inrow + rowpref
    sel = (ikey > T) | ((ikey == T) & (rank <= need))
    nm = sel.astype(jnp.float32)
    nm_out[...] = nm
    tm_out[...] = jnp.tanh(score) * nm


def _tc_pool_a(k, sp0, sp1, innorm, sb, mask):
    return pl.pallas_call(
        functools.partial(_pool_a_body, k),
        out_shape=[
            jax.ShapeDtypeStruct((RB, 128), jnp.float32),
            jax.ShapeDtypeStruct((RB, 128), jnp.float32),
        ],
    )(sp0, sp1, innorm, sb, mask)


def _pool_b_body(k, outfeat, tm, nm, ro_in, featn, ro_out):
    f = outfeat[...] * tm[...]
    featn[...] = f
    avg = jnp.sum(f, axis=0, keepdims=True) / float(k)
    mx = jnp.max(jnp.where(nm[...] > 0, f, NEG), axis=0, keepdims=True)
    ro_out[...] = ro_in[...] + jnp.concatenate([avg, mx], axis=1)


def _tc_pool_b(k, outfeat, tm, nm, ro_in):
    return pl.pallas_call(
        functools.partial(_pool_b_body, k),
        out_shape=[
            jax.ShapeDtypeStruct((NPAD, D), jnp.float32),
            jax.ShapeDtypeStruct((1, 2 * D), jnp.float32),
        ],
    )(outfeat, tm, nm, ro_in)


def _final_body(k, outfeat, tm, nm, ro_in,
                l1W, l1b, l2W, l2b, l3W, l3b, ls_out, h_out):
    f = outfeat[...] * tm[...]
    avg = jnp.sum(f, axis=0, keepdims=True) / float(k)
    mx = jnp.max(jnp.where(nm[...] > 0, f, NEG), axis=0, keepdims=True)
    ro = ro_in[...] + jnp.concatenate([avg, mx], axis=1)
    h1 = jnp.maximum(jnp.dot(ro, l1W[...], preferred_element_type=jnp.float32)
                     + l1b[...], 0.0)
    h2 = jnp.maximum(jnp.dot(h1, l2W[...], preferred_element_type=jnp.float32)
                     + l2b[...], 0.0)
    logits = jnp.dot(h2, l3W[...], preferred_element_type=jnp.float32) + l3b[...]
    m = jnp.max(logits, axis=1, keepdims=True)
    ls_out[...] = logits - m - jnp.log(
        jnp.sum(jnp.exp(logits - m), axis=1, keepdims=True))
    h_out[...] = h2


def _tc_final(k, outfeat, tm, nm, ro_in, l1W, l1b, l2W, l2b, l3W, l3b):
    return pl.pallas_call(
        functools.partial(_final_body, k),
        out_shape=[
            jax.ShapeDtypeStruct((1, OUT_DIM), jnp.float32),
            jax.ShapeDtypeStruct((1, D), jnp.float32),
        ],
    )(outfeat, tm, nm, ro_in, l1W, l1b, l2W, l2b, l3W, l3b)


def kernel(x, edge_index, params):
    src = edge_index[0].astype(jnp.int32)
    dst = edge_index[1].astype(jnp.int32)
    pad = N + (jnp.arange(E_PAD - E, dtype=jnp.int32) % 64)
    srcp = jnp.concatenate([src, pad])
    dstp = jnp.concatenate([dst, pad])
    cnt = jnp.full((NW, 16), NCH, jnp.int32)

    feat = jnp.pad(x, ((0, NPAD - N), (0, 0)))
    mask = jnp.pad(jnp.ones((N, 1), jnp.float32), ((0, NPAD - N), (0, 0)))
    readout = jnp.zeros((1, 2 * D), jnp.float32)

    count = N
    for i in range(3):
        W = params['conv%d_W' % i]
        b = params['conv%d_b' % i][None, :]
        sW = params['score%d_W' % i]
        sb = params['score%d_b' % i][None, :]
        k = int(math.ceil(0.5 * count))

        degp, srcp, dstp, cnt = _sc_degc(mask[:, 0], srcp, dstp, cnt)
        od0 = degp[0, 0][:, None]
        od1 = degp[1, 0][:, None]
        id0 = degp[0, 1][:, None]
        id1 = degp[1, 1][:, None]
        h, innorm, onorm = _tc_norm(od0, od1, id0, id1, mask, feat)
        aggp = _sc_featmp(h, srcp, dstp, cnt)
        out, q = _tc_conv(aggp[0], aggp[1], innorm, onorm, W, b, sW)
        scorep = _sc_scalarmp(q[:, 0], srcp, dstp, cnt)

        sp0 = scorep[0].reshape(RB, 128)
        sp1 = scorep[1].reshape(RB, 128)
        inn2 = innorm.reshape(RB, 128)
        m2 = mask.reshape(RB, 128)
        nm2, tm2 = _tc_pool_a(k, sp0, sp1, inn2, sb, m2)
        nm = nm2.reshape(NPAD, 1)
        tm = tm2.reshape(NPAD, 1)
        if i < 2:
            feat, readout = _tc_pool_b(k, out, tm, nm, readout)
            mask = nm
            count = k
        else:
            return _tc_final(k, out, tm, nm, readout,
                             params['lin1_W'], params['lin1_b'][None, :],
                             params['lin2_W'], params['lin2_b'][None, :],
                             params['lin3_W'], params['lin3_b'][None, :])


# speedup vs baseline: 1.1194x; 1.1194x over previous
import functools
import math

import jax
import jax.numpy as jnp
from jax import lax
from jax.experimental import pallas as pl
from jax.experimental.pallas import tpu as pltpu
from jax.experimental.pallas import tpu_sc as plsc

N = 10000
E = 320000
D = 128
OUT_DIM = 64

NC = 2
NS = 16
NW = NC * NS
CH = 128
NPAD = 10240
NPT = NPAD // NS
E_PAD = ((E + NW * CH * 4 - 1) // (NW * CH * 4)) * (NW * CH * 4)
EPT = E_PAD // NW
NCH = EPT // CH
RB = NPAD // 128
FCH = 64

NEG = float(jnp.finfo(jnp.float32).min)

_MESH = plsc.VectorSubcoreMesh(core_axis_name="c", subcore_axis_name="s")
_SC_PARAMS = pltpu.CompilerParams(needs_layout_passes=False)


def _zero_vec(ref, n):
    for t in range(n // 16):
        ref[pl.ds(t * 16, 16)] = jnp.zeros((16,), jnp.float32)


@functools.partial(
    pl.kernel,
    out_type=[
        jax.ShapeDtypeStruct((NC, 2, NPAD), jnp.float32),
        jax.ShapeDtypeStruct((E_PAD,), jnp.int32),
        jax.ShapeDtypeStruct((E_PAD,), jnp.int32),
        jax.ShapeDtypeStruct((NW, 16), jnp.int32),
    ],
    mesh=_MESH,
    compiler_params=_SC_PARAMS,
    scratch_types=[
        pltpu.VMEM((NPAD,), jnp.float32),
        pltpu.VMEM((2, CH), jnp.int32),
        pltpu.VMEM((2, CH), jnp.int32),
        pltpu.VMEM((2, CH), jnp.int32),
        pltpu.VMEM((2, CH), jnp.int32),
        pltpu.VMEM((2, CH), jnp.float32),
        pltpu.VMEM((2, CH), jnp.float32),
        pltpu.VMEM((2, CH), jnp.float32),
        pltpu.VMEM((2, CH), jnp.float32),
        pltpu.VMEM((EPT + 16,), jnp.int32),
        pltpu.VMEM((EPT + 16,), jnp.int32),
        pltpu.VMEM((16,), jnp.int32),
        pltpu.VMEM((16,), jnp.int32),
        pltpu.VMEM((NPT,), jnp.float32),
        pltpu.VMEM_SHARED((NPAD,), jnp.float32),
        pltpu.VMEM_SHARED((NPAD,), jnp.float32),
        pltpu.SemaphoreType.DMA,
        pltpu.SemaphoreType.DMA,
    ],
)
def _sc_degc(mask_hbm, src_hbm, dst_hbm, cnt_hbm,
             out_hbm, csrc_hbm, cdst_hbm, cnt_out_hbm,
             table_v, sidx0, didx0, sidx1, didx1, vout0, vin0, vout1, vin1,
             csrc_v, cdst_v, cobuf, cibuf, zbuf, odeg_sh, ideg_sh, sem0, sem1):
    c = lax.axis_index("c")
    s = lax.axis_index("s")
    wid = s * NC + c
    pltpu.sync_copy(mask_hbm, table_v)
    pltpu.sync_copy(cnt_hbm.at[wid], cibuf)
    _zero_vec(zbuf, NPT)
    pltpu.sync_copy(zbuf, odeg_sh.at[pl.ds(s * NPT, NPT)])
    pltpu.sync_copy(zbuf, ideg_sh.at[pl.ds(s * NPT, NPT)])
    plsc.subcore_barrier()

    rows_w = cibuf[pl.ds(0, 16)][0]
    G = rows_w // 4
    base_e = wid * EPT

    def load(sidx, didx, row, sem):
        for r in range(2):
            off = base_e + (row + r) * CH
            pltpu.async_copy(src_hbm.at[pl.ds(off, CH)], sidx.at[r], sem)
            pltpu.async_copy(dst_hbm.at[pl.ds(off, CH)], didx.at[r], sem)

    def drain(sidx, didx, sem):
        for r in range(2):
            pltpu.make_async_copy(src_hbm.at[pl.ds(0, CH)], sidx.at[r],
                                  sem).wait()
            pltpu.make_async_copy(dst_hbm.at[pl.ds(0, CH)], didx.at[r],
                                  sem).wait()

    def work(sidx, didx, vout, vin, cnt):
        for r in range(2):
            for t in range(CH // 16):
                sv = sidx[r, pl.ds(t * 16, 16)]
                dv = didx[r, pl.ds(t * 16, 16)]
                vout[r, pl.ds(t * 16, 16)] = plsc.load_gather(table_v, [dv])
                vin[r, pl.ds(t * 16, 16)] = plsc.load_gather(table_v, [sv])
        for r in range(2):
            pltpu.sync_copy(vout.at[r], odeg_sh.at[sidx.at[r]], add=True)
            pltpu.sync_copy(vin.at[r], ideg_sh.at[didx.at[r]], add=True)
        for r in range(2):
            for t in range(CH // 16):
                sv = sidx[r, pl.ds(t * 16, 16)]
                dv = didx[r, pl.ds(t * 16, 16)]
                m = (vout[r, pl.ds(t * 16, 16)]
                     * vin[r, pl.ds(t * 16, 16)]) > 0.0
                plsc.store_compressed(csrc_v.at[pl.ds(cnt, 16)], sv, mask=m)
                plsc.store_compressed(cdst_v.at[pl.ds(cnt, 16)], dv, mask=m)
                cnt = cnt + jnp.sum(m.astype(jnp.int32))
        return cnt

    @pl.when(G >= 1)
    def _():
        load(sidx0, didx0, 0, sem0)

    def body(g, cnt):
        j0 = 4 * g
        load(sidx1, didx1, j0 + 2, sem1)
        drain(sidx0, didx0, sem0)
        cnt = work(sidx0, didx0, vout0, vin0, cnt)

        @pl.when(g + 1 < G)
        def _():
            load(sidx0, didx0, j0 + 4, sem0)

        drain(sidx1, didx1, sem1)
        cnt = work(sidx1, didx1, vout1, vin1, cnt)
        return cnt

    cnt = lax.fori_loop(0, G, body, jnp.int32(0))

    padv = jnp.int32(N) + lax.iota(jnp.int32, 16)
    target = ((cnt + 4 * CH - 1) // (4 * CH)) * (4 * CH)

    def padbody(t, cc):
        @pl.when(cc < target)
        def _():
            csrc_v[pl.ds(cc, 16)] = padv
            cdst_v[pl.ds(cc, 16)] = padv
        return cc + 16

    lax.fori_loop(0, 4 * CH // 16, padbody, cnt)
    cobuf[...] = jnp.broadcast_to(target // CH, (16,)).astype(jnp.int32)
    pltpu.sync_copy(cobuf, cnt_out_hbm.at[wid])
    pltpu.sync_copy(csrc_v.at[pl.ds(0, EPT)], csrc_hbm.at[pl.ds(base_e, EPT)])
    pltpu.sync_copy(cdst_v.at[pl.ds(0, EPT)], cdst_hbm.at[pl.ds(base_e, EPT)])

    plsc.subcore_barrier()
    pltpu.sync_copy(odeg_sh.at[pl.ds(s * NPT, NPT)], zbuf)
    pltpu.sync_copy(zbuf, out_hbm.at[c, 0, pl.ds(s * NPT, NPT)])
    pltpu.sync_copy(ideg_sh.at[pl.ds(s * NPT, NPT)], zbuf)
    pltpu.sync_copy(zbuf, out_hbm.at[c, 1, pl.ds(s * NPT, NPT)])


@functools.partial(
    pl.kernel,
    out_type=jax.ShapeDtypeStruct((NC, NPAD, D), jnp.float32),
    mesh=_MESH,
    compiler_params=_SC_PARAMS,
    scratch_types=[
        pltpu.VMEM((1, CH), jnp.int32),
        pltpu.VMEM((1, CH), jnp.int32),
        pltpu.VMEM((1, CH), jnp.int32),
        pltpu.VMEM((1, CH), jnp.int32),
        pltpu.VMEM((CH, D), jnp.float32),
        pltpu.VMEM((CH, D), jnp.float32),
        pltpu.VMEM((16,), jnp.int32),
        pltpu.VMEM_SHARED((NPAD, D), jnp.float32),
        pltpu.SemaphoreType.DMA,
        pltpu.SemaphoreType.DMA,
    ],
)
def _sc_featmp(h_hbm, src_hbm, dst_hbm, cnt_hbm, out_hbm,
               sidx0, didx0, sidx1, didx1, rows0, rows1, cibuf,
               agg_sh, sem0, sem1):
    c = lax.axis_index("c")
    s = lax.axis_index("s")
    wid = s * NC + c
    pltpu.sync_copy(cnt_hbm.at[wid], cibuf)

    def zrow(r, carry):
        for t in range(D // 16):
            rows0[r, pl.ds(t * 16, 16)] = jnp.zeros((16,), jnp.float32)
        return carry

    lax.fori_loop(0, CH, zrow, 0)
    for q in range(NPT // CH):
        pltpu.sync_copy(rows0, agg_sh.at[pl.ds(s * NPT + q * CH, CH)])
    plsc.subcore_barrier()

    rows_w = cibuf[pl.ds(0, 16)][0]
    G = rows_w // 2
    base_e = wid * EPT

    def load_start(sidx, didx, rows, row, sem):
        off = base_e + row * CH
        pltpu.sync_copy(src_hbm.at[pl.ds(off, CH)], sidx.at[0])
        pltpu.sync_copy(dst_hbm.at[pl.ds(off, CH)], didx.at[0])
        pltpu.async_copy(h_hbm.at[sidx.at[0]], rows, sem)

    @pl.when(G >= 1)
    def _():
        load_start(sidx0, didx0, rows0, 0, sem0)

    def body(g, carry):
        j0 = 2 * g
        load_start(sidx1, didx1, rows1, j0 + 1, sem1)
        pltpu.make_async_copy(h_hbm.at[sidx0.at[0]], rows0, sem0).wait()
        pltpu.sync_copy(rows0, agg_sh.at[didx0.at[0]], add=True)

        @pl.when(g + 1 < G)
        def _():
            load_start(sidx0, didx0, rows0, j0 + 2, sem0)

        pltpu.make_async_copy(h_hbm.at[sidx1.at[0]], rows1, sem1).wait()
        pltpu.sync_copy(rows1, agg_sh.at[didx1.at[0]], add=True)
        return carry

    lax.fori_loop(0, G, body, 0)
    plsc.subcore_barrier()
    for q in range(NPT // CH):
        pltpu.sync_copy(agg_sh.at[pl.ds(s * NPT + q * CH, CH)], rows0)
        pltpu.sync_copy(rows0, out_hbm.at[c, pl.ds(s * NPT + q * CH, CH)])


@functools.partial(
    pl.kernel,
    out_type=jax.ShapeDtypeStruct((NC, NPAD), jnp.float32),
    mesh=_MESH,
    compiler_params=_SC_PARAMS,
    scratch_types=[
        pltpu.VMEM((NPAD,), jnp.float32),
        pltpu.VMEM((2, CH), jnp.int32),
        pltpu.VMEM((2, CH), jnp.int32),
        pltpu.VMEM((2, CH), jnp.int32),
        pltpu.VMEM((2, CH), jnp.int32),
        pltpu.VMEM((2, CH), jnp.float32),
        pltpu.VMEM((2, CH), jnp.float32),
        pltpu.VMEM((16,), jnp.int32),
        pltpu.VMEM((NPT,), jnp.float32),
        pltpu.VMEM_SHARED((NPAD,), jnp.float32),
        pltpu.SemaphoreType.DMA,
        pltpu.SemaphoreType.DMA,
    ],
)
def _sc_scalarmp(val_hbm, src_hbm, dst_hbm, cnt_hbm, out_hbm,
                 table_v, sidx0, didx0, sidx1, didx1, vals0, vals1,
                 cibuf, zbuf, acc_sh, sem0, sem1):
    c = lax.axis_index("c")
    s = lax.axis_index("s")
    wid = s * NC + c
    pltpu.sync_copy(val_hbm, table_v)
    pltpu.sync_copy(cnt_hbm.at[wid], cibuf)
    _zero_vec(zbuf, NPT)
    pltpu.sync_copy(zbuf, acc_sh.at[pl.ds(s * NPT, NPT)])
    plsc.subcore_barrier()

    rows_w = cibuf[pl.ds(0, 16)][0]
    G = rows_w // 4
    base_e = wid * EPT

    def load(sidx, didx, row, sem):
        for r in range(2):
            off = base_e + (row + r) * CH
            pltpu.async_copy(src_hbm.at[pl.ds(off, CH)], sidx.at[r], sem)
            pltpu.async_copy(dst_hbm.at[pl.ds(off, CH)], didx.at[r], sem)

    def drain(sidx, didx, sem):
        for r in range(2):
            pltpu.make_async_copy(src_hbm.at[pl.ds(0, CH)], sidx.at[r],
                                  sem).wait()
            pltpu.make_async_copy(dst_hbm.at[pl.ds(0, CH)], didx.at[r],
                                  sem).wait()

    def work(sidx, didx, vals):
        for r in range(2):
            for t in range(CH // 16):
                sv = sidx[r, pl.ds(t * 16, 16)]
                vals[r, pl.ds(t * 16, 16)] = plsc.load_gather(table_v, [sv])
        for r in range(2):
            pltpu.sync_copy(vals.at[r], acc_sh.at[didx.at[r]], add=True)

    @pl.when(G >= 1)
    def _():
        load(sidx0, didx0, 0, sem0)

    def body(g, carry):
        j0 = 4 * g
        load(sidx1, didx1, j0 + 2, sem1)
        drain(sidx0, didx0, sem0)
        work(sidx0, didx0, vals0)

        @pl.when(g + 1 < G)
        def _():
            load(sidx0, didx0, j0 + 4, sem0)

        drain(sidx1, didx1, sem1)
        work(sidx1, didx1, vals1)
        return carry

    lax.fori_loop(0, G, body, 0)
    plsc.subcore_barrier()
    pltpu.sync_copy(acc_sh.at[pl.ds(s * NPT, NPT)], zbuf)
    pltpu.sync_copy(zbuf, out_hbm.at[c, pl.ds(s * NPT, NPT)])


def _norm_body(od0, od1, id0, id1, mask, feat, h, innorm, onorm):
    od = (od0[...] + od1[...]) * mask[...]
    idg = (id0[...] + id1[...]) * mask[...]
    on = jnp.where(od > 0, lax.rsqrt(jnp.maximum(od, 1e-12)), 0.0)
    inn = jnp.where(idg > 0, lax.rsqrt(jnp.maximum(idg, 1e-12)), 0.0)
    onorm[...] = on
    innorm[...] = inn
    h[...] = feat[...] * on


def _tc_norm(od0, od1, id0, id1, mask, feat):
    return pl.pallas_call(
        _norm_body,
        out_shape=[
            jax.ShapeDtypeStruct((NPAD, D), jnp.float32),
            jax.ShapeDtypeStruct((NPAD, 1), jnp.float32),
            jax.ShapeDtypeStruct((NPAD, 1), jnp.float32),
        ],
    )(od0, od1, id0, id1, mask, feat)


def _conv_body(agg0, agg1, innorm, onorm, W, b, sW, out, q):
    z = (agg0[...] + agg1[...]) * innorm[...]
    o = jnp.maximum(jnp.dot(z, W[...], preferred_element_type=jnp.float32)
                    + b[...], 0.0)
    out[...] = o
    q[...] = jnp.dot(o, sW[...], preferred_element_type=jnp.float32) * onorm[...]


def _tc_conv(agg0, agg1, innorm, onorm, W, b, sW):
    GB = 1280
    g = NPAD // GB
    return pl.pallas_call(
        _conv_body,
        grid=(g,),
        in_specs=[
            pl.BlockSpec((GB, D), lambda i: (i, 0)),
            pl.BlockSpec((GB, D), lambda i: (i, 0)),
            pl.BlockSpec((GB, 1), lambda i: (i, 0)),
            pl.BlockSpec((GB, 1), lambda i: (i, 0)),
            pl.BlockSpec((D, D), lambda i: (0, 0)),
            pl.BlockSpec((1, D), lambda i: (0, 0)),
            pl.BlockSpec((D, 1), lambda i: (0, 0)),
        ],
        out_specs=[
            pl.BlockSpec((GB, D), lambda i: (i, 0)),
            pl.BlockSpec((GB, 1), lambda i: (i, 0)),
        ],
        out_shape=[
            jax.ShapeDtypeStruct((NPAD, D), jnp.float32),
            jax.ShapeDtypeStruct((NPAD, 1), jnp.float32),
        ],
    )(agg0, agg1, innorm, onorm, W, b, sW)


def _pool_a_body(k, sp0, sp1, innorm, sb, mask, nm_out, tm_out):
    score = (sp0[...] + sp1[...]) * innorm[...] + sb[0, 0]
    sm = jnp.where(mask[...] > 0, score, NEG)
    bits = lax.bitcast_convert_type(sm, jnp.int32)
    ikey = jnp.where(bits >= 0, bits, bits ^ jnp.int32(0x7FFFFFFF))
    cnt_nn = jnp.sum((ikey >= 0).astype(jnp.int32))
    x0 = jnp.where(cnt_nn >= k, jnp.int32(0), jnp.int32(-2147483648))

    def body(bit, xx):
        y = xx | (jnp.int32(1) << (30 - bit))
        cnt = jnp.sum((ikey >= y).astype(jnp.int32))
        return jnp.where(cnt >= k, y, xx)

    T = lax.fori_loop(0, 31, body, x0)
    cnt_gt = jnp.sum((ikey > T).astype(jnp.int32))
    need = (k - cnt_gt).astype(jnp.float32)
    eqf = (ikey == T).astype(jnp.float32)
    ia = lax.broadcasted_iota(jnp.int32, (D, D), 0)
    ib = lax.broadcasted_iota(jnp.int32, (D, D), 1)
    ltri = (ia <= ib).astype(jnp.float32)
    ra = lax.broadcasted_iota(jnp.int32, (RB, RB), 0)
    rb = lax.broadcasted_iota(jnp.int32, (RB, RB), 1)
    stri = (rb < ra).astype(jnp.float32)
    inrow = jnp.dot(eqf, ltri, preferred_element_type=jnp.float32)
    rowtot = jnp.sum(eqf, axis=1, keepdims=True)
    rowpref = jnp.dot(stri, rowtot, preferred_element_type=jnp.float32)
    rank = inrow + rowpref
    sel = (ikey > T) | ((ikey == T) & (rank <= need))
    nm = sel.astype(jnp.float32)
    nm_out[...] = nm
    tm_out[...] = jnp.tanh(score) * nm


def _tc_pool_a(k, sp0, sp1, innorm, sb, mask):
    return pl.pallas_call(
        functools.partial(_pool_a_body, k),
        out_shape=[
            jax.ShapeDtypeStruct((RB, 128), jnp.float32),
            jax.ShapeDtypeStruct((RB, 128), jnp.float32),
        ],
    )(sp0, sp1, innorm, sb, mask)


def _pool_b_body(k, outfeat, tm, nm, ro_in, featn, ro_out):
    f = outfeat[...] * tm[...]
    featn[...] = f
    avg = jnp.sum(f, axis=0, keepdims=True) / float(k)
    mx = jnp.max(jnp.where(nm[...] > 0, f, NEG), axis=0, keepdims=True)
    ro_out[...] = ro_in[...] + jnp.concatenate([avg, mx], axis=1)


def _tc_pool_b(k, outfeat, tm, nm, ro_in):
    return pl.pallas_call(
        functools.partial(_pool_b_body, k),
        out_shape=[
            jax.ShapeDtypeStruct((NPAD, D), jnp.float32),
            jax.ShapeDtypeStruct((1, 2 * D), jnp.float32),
        ],
    )(outfeat, tm, nm, ro_in)


def _final_body(k, outfeat, tm, nm, ro_in,
                l1W, l1b, l2W, l2b, l3W, l3b, ls_out, h_out):
    f = outfeat[...] * tm[...]
    avg = jnp.sum(f, axis=0, keepdims=True) / float(k)
    mx = jnp.max(jnp.where(nm[...] > 0, f, NEG), axis=0, keepdims=True)
    ro = ro_in[...] + jnp.concatenate([avg, mx], axis=1)
    h1 = jnp.maximum(jnp.dot(ro, l1W[...], preferred_element_type=jnp.float32)
                     + l1b[...], 0.0)
    h2 = jnp.maximum(jnp.dot(h1, l2W[...], preferred_element_type=jnp.float32)
                     + l2b[...], 0.0)
    logits = jnp.dot(h2, l3W[...], preferred_element_type=jnp.float32) + l3b[...]
    m = jnp.max(logits, axis=1, keepdims=True)
    ls_out[...] = logits - m - jnp.log(
        jnp.sum(jnp.exp(logits - m), axis=1, keepdims=True))
    h_out[...] = h2


def _tc_final(k, outfeat, tm, nm, ro_in, l1W, l1b, l2W, l2b, l3W, l3b):
    return pl.pallas_call(
        functools.partial(_final_body, k),
        out_shape=[
            jax.ShapeDtypeStruct((1, OUT_DIM), jnp.float32),
            jax.ShapeDtypeStruct((1, D), jnp.float32),
        ],
    )(outfeat, tm, nm, ro_in, l1W, l1b, l2W, l2b, l3W, l3b)


def kernel(x, edge_index, params):
    src = edge_index[0].astype(jnp.int32)
    dst = edge_index[1].astype(jnp.int32)
    pad = N + (jnp.arange(E_PAD - E, dtype=jnp.int32) % 64)
    srcp = jnp.concatenate([src, pad])
    dstp = jnp.concatenate([dst, pad])
    cnt = jnp.full((NW, 16), NCH, jnp.int32)

    feat = jnp.pad(x, ((0, NPAD - N), (0, 0)))
    mask = jnp.pad(jnp.ones((N, 1), jnp.float32), ((0, NPAD - N), (0, 0)))
    readout = jnp.zeros((1, 2 * D), jnp.float32)

    count = N
    for i in range(3):
        W = params['conv%d_W' % i]
        b = params['conv%d_b' % i][None, :]
        sW = params['score%d_W' % i]
        sb = params['score%d_b' % i][None, :]
        k = int(math.ceil(0.5 * count))

        degp, srcp, dstp, cnt = _sc_degc(mask[:, 0], srcp, dstp, cnt)
        od0 = degp[0, 0][:, None]
        od1 = degp[1, 0][:, None]
        id0 = degp[0, 1][:, None]
        id1 = degp[1, 1][:, None]
        h, innorm, onorm = _tc_norm(od0, od1, id0, id1, mask, feat)
        aggp = _sc_featmp(h, srcp, dstp, cnt)
        out, q = _tc_conv(aggp[0], aggp[1], innorm, onorm, W, b, sW)
        scorep = _sc_scalarmp(q[:, 0], srcp, dstp, cnt)

        sp0 = scorep[0].reshape(RB, 128)
        sp1 = scorep[1].reshape(RB, 128)
        inn2 = innorm.reshape(RB, 128)
        m2 = mask.reshape(RB, 128)
        nm2, tm2 = _tc_pool_a(k, sp0, sp1, inn2, sb, m2)
        nm = nm2.reshape(NPAD, 1)
        tm = tm2.reshape(NPAD, 1)
        if i < 2:
            feat, readout = _tc_pool_b(k, out, tm, nm, readout)
            mask = nm
            count = k
        else:
            return _tc_final(k, out, tm, nm, readout,
                             params['lin1_W'], params['lin1_b'][None, :],
                             params['lin2_W'], params['lin2_b'][None, :],
                             params['lin3_W'], params['lin3_b'][None, :])


# speedup vs baseline: 1.1660x; 1.0416x over previous
import functools
import math

import jax
import jax.numpy as jnp
from jax import lax
from jax.experimental import pallas as pl
from jax.experimental.pallas import tpu as pltpu
from jax.experimental.pallas import tpu_sc as plsc

N = 10000
E = 320000
D = 128
OUT_DIM = 64

NC = 2
NS = 16
NW = NC * NS
CH = 128
NPAD = 10240
NPT = NPAD // NS
E_PAD = ((E + NW * CH * 4 - 1) // (NW * CH * 4)) * (NW * CH * 4)
EPT = E_PAD // NW
NCH = EPT // CH
RB = NPAD // 128
FCH = 64

NEG = float(jnp.finfo(jnp.float32).min)

_MESH = plsc.VectorSubcoreMesh(core_axis_name="c", subcore_axis_name="s")
_SC_PARAMS = pltpu.CompilerParams(needs_layout_passes=False)


def _zero_vec(ref, n):
    for t in range(n // 16):
        ref[pl.ds(t * 16, 16)] = jnp.zeros((16,), jnp.float32)


@functools.partial(
    pl.kernel,
    out_type=[
        jax.ShapeDtypeStruct((NC, 2, NPAD), jnp.float32),
        jax.ShapeDtypeStruct((E_PAD,), jnp.int32),
        jax.ShapeDtypeStruct((E_PAD,), jnp.int32),
        jax.ShapeDtypeStruct((NW, 16), jnp.int32),
    ],
    mesh=_MESH,
    compiler_params=_SC_PARAMS,
    scratch_types=[
        pltpu.VMEM((NPAD,), jnp.float32),
        pltpu.VMEM((2, CH), jnp.int32),
        pltpu.VMEM((2, CH), jnp.int32),
        pltpu.VMEM((2, CH), jnp.int32),
        pltpu.VMEM((2, CH), jnp.int32),
        pltpu.VMEM((2, CH), jnp.float32),
        pltpu.VMEM((2, CH), jnp.float32),
        pltpu.VMEM((2, CH), jnp.float32),
        pltpu.VMEM((2, CH), jnp.float32),
        pltpu.VMEM((EPT + 16,), jnp.int32),
        pltpu.VMEM((EPT + 16,), jnp.int32),
        pltpu.VMEM((16,), jnp.int32),
        pltpu.VMEM((16,), jnp.int32),
        pltpu.VMEM((NPT,), jnp.float32),
        pltpu.VMEM_SHARED((NPAD,), jnp.float32),
        pltpu.VMEM_SHARED((NPAD,), jnp.float32),
        pltpu.SemaphoreType.DMA,
        pltpu.SemaphoreType.DMA,
    ],
)
def _sc_degc(mask_hbm, src_hbm, dst_hbm, cnt_hbm,
             out_hbm, csrc_hbm, cdst_hbm, cnt_out_hbm,
             table_v, sidx0, didx0, sidx1, didx1, vout0, vin0, vout1, vin1,
             csrc_v, cdst_v, cobuf, cibuf, zbuf, odeg_sh, ideg_sh, sem0, sem1):
    c = lax.axis_index("c")
    s = lax.axis_index("s")
    wid = s * NC + c
    pltpu.sync_copy(mask_hbm, table_v)
    pltpu.sync_copy(cnt_hbm.at[wid], cibuf)
    _zero_vec(zbuf, NPT)
    pltpu.sync_copy(zbuf, odeg_sh.at[pl.ds(s * NPT, NPT)])
    pltpu.sync_copy(zbuf, ideg_sh.at[pl.ds(s * NPT, NPT)])
    plsc.subcore_barrier()

    rows_w = cibuf[pl.ds(0, 16)][0]
    G = rows_w // 4
    base_e = wid * EPT

    def load(sidx, didx, row, sem):
        for r in range(2):
            off = base_e + (row + r) * CH
            pltpu.async_copy(src_hbm.at[pl.ds(off, CH)], sidx.at[r], sem)
            pltpu.async_copy(dst_hbm.at[pl.ds(off, CH)], didx.at[r], sem)

    def drain(sidx, didx, sem):
        for r in range(2):
            pltpu.make_async_copy(src_hbm.at[pl.ds(0, CH)], sidx.at[r],
                                  sem).wait()
            pltpu.make_async_copy(dst_hbm.at[pl.ds(0, CH)], didx.at[r],
                                  sem).wait()

    def work(sidx, didx, vout, vin, cnt):
        for r in range(2):
            for t in range(CH // 16):
                sv = sidx[r, pl.ds(t * 16, 16)]
                dv = didx[r, pl.ds(t * 16, 16)]
                vout[r, pl.ds(t * 16, 16)] = plsc.load_gather(table_v, [dv])
                vin[r, pl.ds(t * 16, 16)] = plsc.load_gather(table_v, [sv])
        for r in range(2):
            pltpu.sync_copy(vout.at[r], odeg_sh.at[sidx.at[r]], add=True)
            pltpu.sync_copy(vin.at[r], ideg_sh.at[didx.at[r]], add=True)
        for r in range(2):
            for t in range(CH // 16):
                sv = sidx[r, pl.ds(t * 16, 16)]
                dv = didx[r, pl.ds(t * 16, 16)]
                m = (vout[r, pl.ds(t * 16, 16)]
                     * vin[r, pl.ds(t * 16, 16)]) > 0.0
                plsc.store_compressed(csrc_v.at[pl.ds(cnt, 16)], sv, mask=m)
                plsc.store_compressed(cdst_v.at[pl.ds(cnt, 16)], dv, mask=m)
                cnt = cnt + jnp.sum(m.astype(jnp.int32))
        return cnt

    @pl.when(G >= 1)
    def _():
        load(sidx0, didx0, 0, sem0)

    def body(g, cnt):
        j0 = 4 * g
        load(sidx1, didx1, j0 + 2, sem1)
        drain(sidx0, didx0, sem0)
        cnt = work(sidx0, didx0, vout0, vin0, cnt)

        @pl.when(g + 1 < G)
        def _():
            load(sidx0, didx0, j0 + 4, sem0)

        drain(sidx1, didx1, sem1)
        cnt = work(sidx1, didx1, vout1, vin1, cnt)
        return cnt

    cnt = lax.fori_loop(0, G, body, jnp.int32(0))

    padv = jnp.int32(N) + lax.iota(jnp.int32, 16)
    target = ((cnt + 4 * CH - 1) // (4 * CH)) * (4 * CH)

    def padbody(t, cc):
        @pl.when(cc < target)
        def _():
            csrc_v[pl.ds(cc, 16)] = padv
            cdst_v[pl.ds(cc, 16)] = padv
        return cc + 16

    lax.fori_loop(0, 4 * CH // 16, padbody, cnt)
    cobuf[...] = jnp.broadcast_to(target // CH, (16,)).astype(jnp.int32)
    pltpu.sync_copy(cobuf, cnt_out_hbm.at[wid])
    pltpu.sync_copy(csrc_v.at[pl.ds(0, EPT)], csrc_hbm.at[pl.ds(base_e, EPT)])
    pltpu.sync_copy(cdst_v.at[pl.ds(0, EPT)], cdst_hbm.at[pl.ds(base_e, EPT)])

    plsc.subcore_barrier()
    pltpu.sync_copy(odeg_sh.at[pl.ds(s * NPT, NPT)], zbuf)
    pltpu.sync_copy(zbuf, out_hbm.at[c, 0, pl.ds(s * NPT, NPT)])
    pltpu.sync_copy(ideg_sh.at[pl.ds(s * NPT, NPT)], zbuf)
    pltpu.sync_copy(zbuf, out_hbm.at[c, 1, pl.ds(s * NPT, NPT)])


@functools.partial(
    pl.kernel,
    out_type=jax.ShapeDtypeStruct((NC, NPAD, D), jnp.float32),
    mesh=_MESH,
    compiler_params=_SC_PARAMS,
    scratch_types=[
        pltpu.VMEM((1, CH), jnp.int32),
        pltpu.VMEM((1, CH), jnp.int32),
        pltpu.VMEM((1, CH), jnp.int32),
        pltpu.VMEM((1, CH), jnp.int32),
        pltpu.VMEM((1, CH), jnp.int32),
        pltpu.VMEM((1, CH), jnp.int32),
        pltpu.VMEM((1, CH), jnp.int32),
        pltpu.VMEM((1, CH), jnp.int32),
        pltpu.VMEM((CH, D), jnp.float32),
        pltpu.VMEM((CH, D), jnp.float32),
        pltpu.VMEM((16,), jnp.int32),
        pltpu.VMEM_SHARED((NPAD, D), jnp.float32),
        pltpu.SemaphoreType.DMA,
        pltpu.SemaphoreType.DMA,
        pltpu.SemaphoreType.DMA,
        pltpu.SemaphoreType.DMA,
        pltpu.SemaphoreType.DMA,
        pltpu.SemaphoreType.DMA,
    ],
)
def _sc_featmp(h_hbm, src_hbm, dst_hbm, cnt_hbm, out_hbm,
               si0, si1, si2, si3, di0, di1, di2, di3, rows0, rows1, cibuf,
               agg_sh, ix0, ix1, ix2, ix3, sga, sgb):
    c = lax.axis_index("c")
    s = lax.axis_index("s")
    wid = s * NC + c
    pltpu.sync_copy(cnt_hbm.at[wid], cibuf)
    SI = [si0, si1, si2, si3]
    DI = [di0, di1, di2, di3]
    IX = [ix0, ix1, ix2, ix3]
    RW = [rows0, rows1]
    SG = [sga, sgb]

    def zrow(r, carry):
        for t in range(D // 16):
            rows0[r, pl.ds(t * 16, 16)] = jnp.zeros((16,), jnp.float32)
        return carry

    lax.fori_loop(0, CH, zrow, 0)
    for q in range(NPT // CH):
        pltpu.sync_copy(rows0, agg_sh.at[pl.ds(s * NPT + q * CH, CH)])
    plsc.subcore_barrier()

    rows_w = cibuf[pl.ds(0, 16)][0]
    G4 = rows_w // 4
    base_e = wid * EPT

    def idx_start(k, j):
        off = base_e + j * CH
        pltpu.async_copy(src_hbm.at[pl.ds(off, CH)], SI[k].at[0], IX[k])
        pltpu.async_copy(dst_hbm.at[pl.ds(off, CH)], DI[k].at[0], IX[k])

    def idx_wait(k):
        pltpu.make_async_copy(src_hbm.at[pl.ds(0, CH)], SI[k].at[0],
                              IX[k]).wait()
        pltpu.make_async_copy(dst_hbm.at[pl.ds(0, CH)], DI[k].at[0],
                              IX[k]).wait()

    def gather_start(k, b):
        pltpu.async_copy(h_hbm.at[SI[k].at[0]], RW[b], SG[b])

    def gather_wait_scatter(k, b):
        pltpu.make_async_copy(h_hbm.at[SI[k].at[0]], RW[b], SG[b]).wait()
        pltpu.sync_copy(RW[b], agg_sh.at[DI[k].at[0]], add=True)

    @pl.when(G4 >= 1)
    def _():
        for k in range(4):
            idx_start(k, k)

    def body(g, carry):
        j0 = 4 * g
        idx_wait(0)
        gather_start(0, 0)
        idx_wait(1)
        gather_start(1, 1)
        gather_wait_scatter(0, 0)

        @pl.when(g + 1 < G4)
        def _():
            idx_start(0, j0 + 4)

        idx_wait(2)
        gather_start(2, 0)
        gather_wait_scatter(1, 1)

        @pl.when(g + 1 < G4)
        def _():
            idx_start(1, j0 + 5)

        idx_wait(3)
        gather_start(3, 1)
        gather_wait_scatter(2, 0)

        @pl.when(g + 1 < G4)
        def _():
            idx_start(2, j0 + 6)

        gather_wait_scatter(3, 1)

        @pl.when(g + 1 < G4)
        def _():
            idx_start(3, j0 + 7)

        return carry

    lax.fori_loop(0, G4, body, 0)
    plsc.subcore_barrier()
    for q in range(NPT // CH):
        pltpu.sync_copy(agg_sh.at[pl.ds(s * NPT + q * CH, CH)], rows0)
        pltpu.sync_copy(rows0, out_hbm.at[c, pl.ds(s * NPT + q * CH, CH)])


@functools.partial(
    pl.kernel,
    out_type=jax.ShapeDtypeStruct((NC, NPAD), jnp.float32),
    mesh=_MESH,
    compiler_params=_SC_PARAMS,
    scratch_types=[
        pltpu.VMEM((NPAD,), jnp.float32),
        pltpu.VMEM((2, CH), jnp.int32),
        pltpu.VMEM((2, CH), jnp.int32),
        pltpu.VMEM((2, CH), jnp.int32),
        pltpu.VMEM((2, CH), jnp.int32),
        pltpu.VMEM((2, CH), jnp.float32),
        pltpu.VMEM((2, CH), jnp.float32),
        pltpu.VMEM((16,), jnp.int32),
        pltpu.VMEM((NPT,), jnp.float32),
        pltpu.VMEM_SHARED((NPAD,), jnp.float32),
        pltpu.SemaphoreType.DMA,
        pltpu.SemaphoreType.DMA,
    ],
)
def _sc_scalarmp(val_hbm, src_hbm, dst_hbm, cnt_hbm, out_hbm,
                 table_v, sidx0, didx0, sidx1, didx1, vals0, vals1,
                 cibuf, zbuf, acc_sh, sem0, sem1):
    c = lax.axis_index("c")
    s = lax.axis_index("s")
    wid = s * NC + c
    pltpu.sync_copy(val_hbm, table_v)
    pltpu.sync_copy(cnt_hbm.at[wid], cibuf)
    _zero_vec(zbuf, NPT)
    pltpu.sync_copy(zbuf, acc_sh.at[pl.ds(s * NPT, NPT)])
    plsc.subcore_barrier()

    rows_w = cibuf[pl.ds(0, 16)][0]
    G = rows_w // 4
    base_e = wid * EPT

    def load(sidx, didx, row, sem):
        for r in range(2):
            off = base_e + (row + r) * CH
            pltpu.async_copy(src_hbm.at[pl.ds(off, CH)], sidx.at[r], sem)
            pltpu.async_copy(dst_hbm.at[pl.ds(off, CH)], didx.at[r], sem)

    def drain(sidx, didx, sem):
        for r in range(2):
            pltpu.make_async_copy(src_hbm.at[pl.ds(0, CH)], sidx.at[r],
                                  sem).wait()
            pltpu.make_async_copy(dst_hbm.at[pl.ds(0, CH)], didx.at[r],
                                  sem).wait()

    def work(sidx, didx, vals):
        for r in range(2):
            for t in range(CH // 16):
                sv = sidx[r, pl.ds(t * 16, 16)]
                vals[r, pl.ds(t * 16, 16)] = plsc.load_gather(table_v, [sv])
        for r in range(2):
            pltpu.sync_copy(vals.at[r], acc_sh.at[didx.at[r]], add=True)

    @pl.when(G >= 1)
    def _():
        load(sidx0, didx0, 0, sem0)

    def body(g, carry):
        j0 = 4 * g
        load(sidx1, didx1, j0 + 2, sem1)
        drain(sidx0, didx0, sem0)
        work(sidx0, didx0, vals0)

        @pl.when(g + 1 < G)
        def _():
            load(sidx0, didx0, j0 + 4, sem0)

        drain(sidx1, didx1, sem1)
        work(sidx1, didx1, vals1)
        return carry

    lax.fori_loop(0, G, body, 0)
    plsc.subcore_barrier()
    pltpu.sync_copy(acc_sh.at[pl.ds(s * NPT, NPT)], zbuf)
    pltpu.sync_copy(zbuf, out_hbm.at[c, pl.ds(s * NPT, NPT)])


def _norm_body(od0, od1, id0, id1, mask, feat, h, innorm, onorm):
    od = (od0[...] + od1[...]) * mask[...]
    idg = (id0[...] + id1[...]) * mask[...]
    on = jnp.where(od > 0, lax.rsqrt(jnp.maximum(od, 1e-12)), 0.0)
    inn = jnp.where(idg > 0, lax.rsqrt(jnp.maximum(idg, 1e-12)), 0.0)
    onorm[...] = on
    innorm[...] = inn
    h[...] = feat[...] * on


def _tc_norm(od0, od1, id0, id1, mask, feat):
    return pl.pallas_call(
        _norm_body,
        out_shape=[
            jax.ShapeDtypeStruct((NPAD, D), jnp.float32),
            jax.ShapeDtypeStruct((NPAD, 1), jnp.float32),
            jax.ShapeDtypeStruct((NPAD, 1), jnp.float32),
        ],
    )(od0, od1, id0, id1, mask, feat)


def _conv_body(agg0, agg1, innorm, onorm, W, b, sW, out, q):
    z = (agg0[...] + agg1[...]) * innorm[...]
    o = jnp.maximum(jnp.dot(z, W[...], preferred_element_type=jnp.float32)
                    + b[...], 0.0)
    out[...] = o
    q[...] = jnp.dot(o, sW[...], preferred_element_type=jnp.float32) * onorm[...]


def _tc_conv(agg0, agg1, innorm, onorm, W, b, sW):
    GB = 1280
    g = NPAD // GB
    return pl.pallas_call(
        _conv_body,
        grid=(g,),
        in_specs=[
            pl.BlockSpec((GB, D), lambda i: (i, 0)),
            pl.BlockSpec((GB, D), lambda i: (i, 0)),
            pl.BlockSpec((GB, 1), lambda i: (i, 0)),
            pl.BlockSpec((GB, 1), lambda i: (i, 0)),
            pl.BlockSpec((D, D), lambda i: (0, 0)),
            pl.BlockSpec((1, D), lambda i: (0, 0)),
            pl.BlockSpec((D, 1), lambda i: (0, 0)),
        ],
        out_specs=[
            pl.BlockSpec((GB, D), lambda i: (i, 0)),
            pl.BlockSpec((GB, 1), lambda i: (i, 0)),
        ],
        out_shape=[
            jax.ShapeDtypeStruct((NPAD, D), jnp.float32),
            jax.ShapeDtypeStruct((NPAD, 1), jnp.float32),
        ],
    )(agg0, agg1, innorm, onorm, W, b, sW)


def _pool_a_body(k, sp0, sp1, innorm, sb, mask, nm_out, tm_out):
    score = (sp0[...] + sp1[...]) * innorm[...] + sb[0, 0]
    sm = jnp.where(mask[...] > 0, score, NEG)
    bits = lax.bitcast_convert_type(sm, jnp.int32)
    ikey = jnp.where(bits >= 0, bits, bits ^ jnp.int32(0x7FFFFFFF))
    cnt_nn = jnp.sum((ikey >= 0).astype(jnp.int32))
    x0 = jnp.where(cnt_nn >= k, jnp.int32(0), jnp.int32(-2147483648))

    def body(bit, xx):
        y = xx | (jnp.int32(1) << (30 - bit))
        cnt = jnp.sum((ikey >= y).astype(jnp.int32))
        return jnp.where(cnt >= k, y, xx)

    T = lax.fori_loop(0, 31, body, x0)
    cnt_gt = jnp.sum((ikey > T).astype(jnp.int32))
    need = (k - cnt_gt).astype(jnp.float32)
    eqf = (ikey == T).astype(jnp.float32)
    ia = lax.broadcasted_iota(jnp.int32, (D, D), 0)
    ib = lax.broadcasted_iota(jnp.int32, (D, D), 1)
    ltri = (ia <= ib).astype(jnp.float32)
    ra = lax.broadcasted_iota(jnp.int32, (RB, RB), 0)
    rb = lax.broadcasted_iota(jnp.int32, (RB, RB), 1)
    stri = (rb < ra).astype(jnp.float32)
    inrow = jnp.dot(eqf, ltri, preferred_element_type=jnp.float32)
    rowtot = jnp.sum(eqf, axis=1, keepdims=True)
    rowpref = jnp.dot(stri, rowtot, preferred_element_type=jnp.float32)
    rank = inrow + rowpref
    sel = (ikey > T) | ((ikey == T) & (rank <= need))
    nm = sel.astype(jnp.float32)
    nm_out[...] = nm
    tm_out[...] = jnp.tanh(score) * nm


def _tc_pool_a(k, sp0, sp1, innorm, sb, mask):
    return pl.pallas_call(
        functools.partial(_pool_a_body, k),
        out_shape=[
            jax.ShapeDtypeStruct((RB, 128), jnp.float32),
            jax.ShapeDtypeStruct((RB, 128), jnp.float32),
        ],
    )(sp0, sp1, innorm, sb, mask)


def _pool_b_body(k, outfeat, tm, nm, ro_in, featn, ro_out):
    f = outfeat[...] * tm[...]
    featn[...] = f
    avg = jnp.sum(f, axis=0, keepdims=True) / float(k)
    mx = jnp.max(jnp.where(nm[...] > 0, f, NEG), axis=0, keepdims=True)
    ro_out[...] = ro_in[...] + jnp.concatenate([avg, mx], axis=1)


def _tc_pool_b(k, outfeat, tm, nm, ro_in):
    return pl.pallas_call(
        functools.partial(_pool_b_body, k),
        out_shape=[
            jax.ShapeDtypeStruct((NPAD, D), jnp.float32),
            jax.ShapeDtypeStruct((1, 2 * D), jnp.float32),
        ],
    )(outfeat, tm, nm, ro_in)


def _final_body(k, outfeat, tm, nm, ro_in,
                l1W, l1b, l2W, l2b, l3W, l3b, ls_out, h_out):
    f = outfeat[...] * tm[...]
    avg = jnp.sum(f, axis=0, keepdims=True) / float(k)
    mx = jnp.max(jnp.where(nm[...] > 0, f, NEG), axis=0, keepdims=True)
    ro = ro_in[...] + jnp.concatenate([avg, mx], axis=1)
    h1 = jnp.maximum(jnp.dot(ro, l1W[...], preferred_element_type=jnp.float32)
                     + l1b[...], 0.0)
    h2 = jnp.maximum(jnp.dot(h1, l2W[...], preferred_element_type=jnp.float32)
                     + l2b[...], 0.0)
    logits = jnp.dot(h2, l3W[...], preferred_element_type=jnp.float32) + l3b[...]
    m = jnp.max(logits, axis=1, keepdims=True)
    ls_out[...] = logits - m - jnp.log(
        jnp.sum(jnp.exp(logits - m), axis=1, keepdims=True))
    h_out[...] = h2


def _tc_final(k, outfeat, tm, nm, ro_in, l1W, l1b, l2W, l2b, l3W, l3b):
    return pl.pallas_call(
        functools.partial(_final_body, k),
        out_shape=[
            jax.ShapeDtypeStruct((1, OUT_DIM), jnp.float32),
            jax.ShapeDtypeStruct((1, D), jnp.float32),
        ],
    )(outfeat, tm, nm, ro_in, l1W, l1b, l2W, l2b, l3W, l3b)


def kernel(x, edge_index, params):
    src = edge_index[0].astype(jnp.int32)
    dst = edge_index[1].astype(jnp.int32)
    pad = N + (jnp.arange(E_PAD - E, dtype=jnp.int32) % 64)
    srcp = jnp.concatenate([src, pad])
    dstp = jnp.concatenate([dst, pad])
    cnt = jnp.full((NW, 16), NCH, jnp.int32)

    feat = jnp.pad(x, ((0, NPAD - N), (0, 0)))
    mask = jnp.pad(jnp.ones((N, 1), jnp.float32), ((0, NPAD - N), (0, 0)))
    readout = jnp.zeros((1, 2 * D), jnp.float32)

    count = N
    for i in range(3):
        W = params['conv%d_W' % i]
        b = params['conv%d_b' % i][None, :]
        sW = params['score%d_W' % i]
        sb = params['score%d_b' % i][None, :]
        k = int(math.ceil(0.5 * count))

        degp, srcp, dstp, cnt = _sc_degc(mask[:, 0], srcp, dstp, cnt)
        od0 = degp[0, 0][:, None]
        od1 = degp[1, 0][:, None]
        id0 = degp[0, 1][:, None]
        id1 = degp[1, 1][:, None]
        h, innorm, onorm = _tc_norm(od0, od1, id0, id1, mask, feat)
        aggp = _sc_featmp(h, srcp, dstp, cnt)
        out, q = _tc_conv(aggp[0], aggp[1], innorm, onorm, W, b, sW)
        scorep = _sc_scalarmp(q[:, 0], srcp, dstp, cnt)

        sp0 = scorep[0].reshape(RB, 128)
        sp1 = scorep[1].reshape(RB, 128)
        inn2 = innorm.reshape(RB, 128)
        m2 = mask.reshape(RB, 128)
        nm2, tm2 = _tc_pool_a(k, sp0, sp1, inn2, sb, m2)
        nm = nm2.reshape(NPAD, 1)
        tm = tm2.reshape(NPAD, 1)
        if i < 2:
            feat, readout = _tc_pool_b(k, out, tm, nm, readout)
            mask = nm
            count = k
        else:
            return _tc_final(k, out, tm, nm, readout,
                             params['lin1_W'], params['lin1_b'][None, :],
                             params['lin2_W'], params['lin2_b'][None, :],
                             params['lin3_W'], params['lin3_b'][None, :])
